# baseline (speedup 1.0000x reference)
import functools

import jax
import jax.numpy as jnp
from jax import lax
from jax.experimental import pallas as pl
from jax.experimental.pallas import tpu as pltpu
from jax.experimental.pallas import tpu_sc as plsc

B, E, H = 8, 16384, 256
MR = MC = MT = 2048
K_R, K_CT = 6, 4
M_RF = B * MR
M_CTF = B * (MC + MT)
NW = 32
CHUNK = 128
HW = H // 2


def _pack_body(x_ref, out_ref):
    x = x_ref[...]
    lo = lax.bitcast_convert_type(x[:, :HW].astype(jnp.bfloat16), jnp.uint16)
    hi = lax.bitcast_convert_type(x[:, HW:].astype(jnp.bfloat16), jnp.uint16)
    w = lo.astype(jnp.uint32) | (hi.astype(jnp.uint32) << 16)
    out_ref[...] = lax.bitcast_convert_type(w, jnp.int32)


def _pack_table(table):
    n = table.shape[0]
    rb = 4096
    return pl.pallas_call(
        _pack_body,
        grid=(n // rb,),
        in_specs=[pl.BlockSpec((rb, H), lambda i: (i, 0))],
        out_specs=pl.BlockSpec((rb, HW), lambda i: (i, 0)),
        out_shape=jax.ShapeDtypeStruct((n, HW), jnp.int32),
    )(table)


def _sc_gather(table, idx, k, m_rows, half, n_halves):
    mesh = plsc.VectorSubcoreMesh(core_axis_name="c", subcore_axis_name="s")
    m_half = m_rows // n_halves
    per_w = m_half // NW
    width = table.shape[1]
    dt = table.dtype

    @functools.partial(
        pl.kernel,
        mesh=mesh,
        out_type=tuple(
            jax.ShapeDtypeStruct((m_half, width), dt) for _ in range(k)
        ),
        scratch_types=[
            pltpu.VMEM((per_w,), jnp.int32),
            pltpu.VMEM((CHUNK, width), dt),
            pltpu.VMEM((CHUNK, width), dt),
            pltpu.SemaphoreType.DMA,
            pltpu.SemaphoreType.DMA,
        ],
    )
    def gather_kernel(table_hbm, idx_hbm, *refs):
        out_refs = refs[:k]
        idx_v, buf0, buf1, sem0, sem1 = refs[k:]
        wid = lax.axis_index("s") * 2 + lax.axis_index("c")

        def run(idx_base, out_hbm):
            base = wid * per_w
            nchunk = per_w // CHUNK
            pltpu.sync_copy(
                idx_hbm.at[pl.ds(idx_base + base, per_w)],
                idx_v.at[pl.ds(0, per_w)],
            )

            def issue(c, buf, sem):
                pltpu.async_copy(
                    table_hbm.at[idx_v.at[pl.ds(c * CHUNK, CHUNK)]], buf, sem
                )

            def drain(c, buf, sem):
                pltpu.make_async_copy(
                    table_hbm.at[idx_v.at[pl.ds(c * CHUNK, CHUNK)]], buf, sem
                ).wait()
                pltpu.sync_copy(buf, out_hbm.at[pl.ds(base + c * CHUNK, CHUNK)])

            issue(0, buf0, sem0)

            def body(p, carry):
                c0 = p * 2
                issue(c0 + 1, buf1, sem1)
                drain(c0, buf0, sem0)

                @pl.when(c0 + 2 < nchunk)
                def _issue_next():
                    issue(c0 + 2, buf0, sem0)

                drain(c0 + 1, buf1, sem1)
                return carry

            lax.fori_loop(0, nchunk // 2, body, 0)

        for j in range(k):
            run(j * m_rows + half * m_half, out_refs[j])

    return gather_kernel(table, idx)


def _mk_mlp_body(k, packed, row_block):
    def body(*refs):
        pid = pl.program_id(0)
        x_refs = refs[:k]
        (w1_ref, b1_ref, w2_ref, b2_ref,
         wa1_ref, ba1_ref, wa2_ref, ba2_ref, wa3_ref, ba3_ref,
         wa4t_ref, ba4_ref, out_ref) = refs[k:]
        f32 = jnp.float32
        bf16 = jnp.bfloat16

        def lin(h, w_ref, b_ref):
            return jnp.dot(h, w_ref[...], preferred_element_type=f32) + b_ref[...]

        s = b1_ref[...].astype(f32)
        for j in range(k):
            w = x_refs[j][...]
            if packed:
                lo = lax.bitcast_convert_type(w << 16, f32)
                hi = lax.bitcast_convert_type(w & jnp.int32(-65536), f32)
                xj = jnp.concatenate([lo, hi], axis=1).astype(bf16)
            else:
                xj = w.astype(bf16)
            s = s + jnp.dot(xj, w1_ref[j * H:(j + 1) * H, :],
                            preferred_element_type=f32)
        h = jnp.maximum(s, 0.0).astype(bf16)
        m = lin(h, w2_ref, b2_ref).astype(bf16)
        h = jnp.maximum(lin(m, wa1_ref, ba1_ref), 0.0).astype(bf16)
        h = jnp.maximum(lin(h, wa2_ref, ba2_ref), 0.0).astype(bf16)
        h = jnp.maximum(lin(h, wa3_ref, ba3_ref), 0.0).astype(bf16)
        res = lax.dot_general(wa4t_ref[...], h, (((1,), (1,)), ((), ())),
                              preferred_element_type=f32) + ba4_ref[...]
        sub = MR // row_block
        out_ref[pl.ds(pid // sub, 1), pl.ds((pid % sub) * row_block, row_block)] = res
    return body


def _mlp_stack(xs, w1, b1, w2, b2, wa1, ba1, wa2, ba2, wa3, ba3, wa4, ba4,
               row_block, packed):
    k = len(xs)
    n, width = xs[0].shape
    grid = (n // row_block,)
    rpb = row_block // MR
    fixed = lambda i: (0, 0)
    out = pl.pallas_call(
        _mk_mlp_body(k, packed, row_block),
        grid=grid,
        in_specs=[pl.BlockSpec((row_block, width), lambda i: (i, 0))] * k + [
            pl.BlockSpec((k * H, H), fixed),
            pl.BlockSpec((1, H), fixed),
            pl.BlockSpec((H, H), fixed),
            pl.BlockSpec((1, H), fixed),
            pl.BlockSpec((H, H), fixed),
            pl.BlockSpec((1, H), fixed),
            pl.BlockSpec((H, H), fixed),
            pl.BlockSpec((1, H), fixed),
            pl.BlockSpec((H, H), fixed),
            pl.BlockSpec((1, H), fixed),
            pl.BlockSpec((1, H), fixed),
            pl.BlockSpec((1, 1), fixed),
        ],
        out_specs=pl.BlockSpec((n // MR, MR), fixed),
        out_shape=jax.ShapeDtypeStruct((n // MR, MR), jnp.float32),
    )(*xs, w1, b1, w2, b2, wa1, ba1, wa2, ba2, wa3, ba3, wa4, ba4)
    return out


def kernel(e_emb, reloc_idx, cross_idx, twoopt_idx,
           Wr1, br1, Wr2, br2,
           Wc1, bc1, Wc2, bc2,
           Wa1, ba1, Wa2, ba2, Wa3, ba3, Wa4, ba4):
    bf16 = jnp.bfloat16
    offs = (jnp.arange(B, dtype=jnp.int32) * E)[:, None, None]
    ridx = jnp.transpose(reloc_idx.astype(jnp.int32) + offs, (2, 0, 1)).reshape(-1)
    cidx = jnp.transpose(cross_idx.astype(jnp.int32) + offs, (2, 0, 1)).reshape(K_CT, -1)
    tidx = jnp.transpose(twoopt_idx.astype(jnp.int32) + offs, (2, 0, 1)).reshape(K_CT, -1)
    ctidx = jnp.concatenate([cidx, tidx], axis=1).reshape(-1)

    table_pk = _pack_table(e_emb.reshape(B * E, H))
    xs_c = _sc_gather(table_pk, ctidx, K_CT, M_CTF, 0, 2)
    xs_t = _sc_gather(table_pk, ctidx, K_CT, M_CTF, 1, 2)
    xs_r1 = _sc_gather(table_pk, ridx, K_R, M_RF, 0, 2)
    xs_r2 = _sc_gather(table_pk, ridx, K_R, M_RF, 1, 2)

    r1 = lambda v: v.reshape(1, -1)
    wb = lambda w: w.astype(bf16)
    args_r = (wb(Wr1), r1(br1), wb(Wr2), r1(br2),
              wb(Wa1), r1(ba1), wb(Wa2), r1(ba2), wb(Wa3), r1(ba3),
              wb(Wa4).T, r1(ba4))
    args_c = (wb(Wc1), r1(bc1), wb(Wc2), r1(bc2),
              wb(Wa1), r1(ba1), wb(Wa2), r1(ba2), wb(Wa3), r1(ba3),
              wb(Wa4).T, r1(ba4))
    lc = _mlp_stack(xs_c, *args_c, row_block=2048, packed=True)
    lt = _mlp_stack(xs_t, *args_c, row_block=2048, packed=True)
    lr1 = _mlp_stack(xs_r1, *args_r, row_block=1024, packed=True)
    lr2 = _mlp_stack(xs_r2, *args_r, row_block=1024, packed=True)

    lr = jnp.concatenate([lr1, lr2], axis=0)
    return jnp.concatenate([lr, lc, lt], axis=1)


# speedup vs baseline: 1.0023x; 1.0023x over previous
import functools

import jax
import jax.numpy as jnp
from jax import lax
from jax.experimental import pallas as pl
from jax.experimental.pallas import tpu as pltpu
from jax.experimental.pallas import tpu_sc as plsc

B, E, H = 8, 16384, 256
MR = MC = MT = 2048
K_R, K_CT = 6, 4
M_RF = B * MR
M_CTF = B * (MC + MT)
NW = 32
CHUNK = 128
HW = H // 2


def _pack_body(x_ref, out_ref):
    x = x_ref[...]
    lo = lax.bitcast_convert_type(x[:, :HW].astype(jnp.bfloat16), jnp.uint16)
    hi = lax.bitcast_convert_type(x[:, HW:].astype(jnp.bfloat16), jnp.uint16)
    w = lo.astype(jnp.uint32) | (hi.astype(jnp.uint32) << 16)
    out_ref[...] = lax.bitcast_convert_type(w, jnp.int32)


def _pack_table(table):
    n = table.shape[0]
    rb = 4096
    return pl.pallas_call(
        _pack_body,
        grid=(n // rb,),
        in_specs=[pl.BlockSpec((rb, H), lambda i: (i, 0))],
        out_specs=pl.BlockSpec((rb, HW), lambda i: (i, 0)),
        out_shape=jax.ShapeDtypeStruct((n, HW), jnp.int32),
    )(table)


def _sc_gather(table, idx, k, m_rows, half, n_halves):
    mesh = plsc.VectorSubcoreMesh(core_axis_name="c", subcore_axis_name="s")
    m_half = m_rows // n_halves
    per_w = m_half // NW
    width = table.shape[1]
    dt = table.dtype

    @functools.partial(
        pl.kernel,
        mesh=mesh,
        out_type=tuple(
            jax.ShapeDtypeStruct((m_half, width), dt) for _ in range(k)
        ),
        scratch_types=[
            pltpu.VMEM((per_w,), jnp.int32),
            pltpu.VMEM((CHUNK, width), dt),
            pltpu.VMEM((CHUNK, width), dt),
            pltpu.SemaphoreType.DMA,
            pltpu.SemaphoreType.DMA,
        ],
    )
    def gather_kernel(table_hbm, idx_hbm, *refs):
        out_refs = refs[:k]
        idx_v, buf0, buf1, sem0, sem1 = refs[k:]
        wid = lax.axis_index("s") * 2 + lax.axis_index("c")

        def run(idx_base, out_hbm):
            base = wid * per_w
            nchunk = per_w // CHUNK
            pltpu.sync_copy(
                idx_hbm.at[pl.ds(idx_base + base, per_w)],
                idx_v.at[pl.ds(0, per_w)],
            )

            def issue(c, buf, sem):
                pltpu.async_copy(
                    table_hbm.at[idx_v.at[pl.ds(c * CHUNK, CHUNK)]], buf, sem
                )

            def drain(c, buf, sem):
                pltpu.make_async_copy(
                    table_hbm.at[idx_v.at[pl.ds(c * CHUNK, CHUNK)]], buf, sem
                ).wait()
                pltpu.sync_copy(buf, out_hbm.at[pl.ds(base + c * CHUNK, CHUNK)])

            issue(0, buf0, sem0)

            def body(p, carry):
                c0 = p * 2
                issue(c0 + 1, buf1, sem1)
                drain(c0, buf0, sem0)

                @pl.when(c0 + 2 < nchunk)
                def _issue_next():
                    issue(c0 + 2, buf0, sem0)

                drain(c0 + 1, buf1, sem1)
                return carry

            lax.fori_loop(0, nchunk // 2, body, 0)

        for j in range(k):
            run(j * m_rows + half * m_half, out_refs[j])

    return gather_kernel(table, idx)


def _mk_mlp_body(k, packed, row_block):
    def body(*refs):
        pid = pl.program_id(0)
        x_refs = refs[:k]
        (w1_ref, b1_ref, wcat_ref, bcat_ref,
         wa4t_ref, ba4_ref, out_ref) = refs[k:]
        f32 = jnp.float32
        bf16 = jnp.bfloat16

        def lin(h, i):
            return (jnp.dot(h, wcat_ref[i * H:(i + 1) * H, :],
                            preferred_element_type=f32)
                    + bcat_ref[pl.ds(i, 1), :])

        s = b1_ref[...].astype(f32)
        for j in range(k):
            w = x_refs[j][...]
            if packed:
                lo = lax.bitcast_convert_type(w << 16, f32)
                hi = lax.bitcast_convert_type(w & jnp.int32(-65536), f32)
                xj = jnp.concatenate([lo, hi], axis=1).astype(bf16)
            else:
                xj = w.astype(bf16)
            s = s + jnp.dot(xj, w1_ref[j * H:(j + 1) * H, :],
                            preferred_element_type=f32)
        h = jnp.maximum(s, 0.0).astype(bf16)
        m = lin(h, 0).astype(bf16)
        h = jnp.maximum(lin(m, 1), 0.0).astype(bf16)
        h = jnp.maximum(lin(h, 2), 0.0).astype(bf16)
        h = jnp.maximum(lin(h, 3), 0.0).astype(bf16)
        res = lax.dot_general(wa4t_ref[...], h, (((1,), (1,)), ((), ())),
                              preferred_element_type=f32) + ba4_ref[...]
        sub = MR // row_block
        out_ref[pl.ds(pid // sub, 1), pl.ds((pid % sub) * row_block, row_block)] = res
    return body


def _mlp_stack(xs, w1, b1, wcat, bcat, wa4t, ba4, row_block, packed):
    k = len(xs)
    n, width = xs[0].shape
    grid = (n // row_block,)
    rpb = row_block // MR
    fixed = lambda i: (0, 0)
    out = pl.pallas_call(
        _mk_mlp_body(k, packed, row_block),
        grid=grid,
        in_specs=[pl.BlockSpec((row_block, width), lambda i: (i, 0))] * k + [
            pl.BlockSpec((k * H, H), fixed),
            pl.BlockSpec((1, H), fixed),
            pl.BlockSpec((4 * H, H), fixed),
            pl.BlockSpec((4, H), fixed),
            pl.BlockSpec((1, H), fixed),
            pl.BlockSpec((1, 1), fixed),
        ],
        out_specs=pl.BlockSpec((n // MR, MR), fixed),
        out_shape=jax.ShapeDtypeStruct((n // MR, MR), jnp.float32),
    )(*xs, w1, b1, wcat, bcat, wa4t, ba4)
    return out


def kernel(e_emb, reloc_idx, cross_idx, twoopt_idx,
           Wr1, br1, Wr2, br2,
           Wc1, bc1, Wc2, bc2,
           Wa1, ba1, Wa2, ba2, Wa3, ba3, Wa4, ba4):
    bf16 = jnp.bfloat16
    offs = (jnp.arange(B, dtype=jnp.int32) * E)[:, None, None]
    ridx = jnp.transpose(reloc_idx.astype(jnp.int32) + offs, (2, 0, 1)).reshape(-1)
    cidx = jnp.transpose(cross_idx.astype(jnp.int32) + offs, (2, 0, 1)).reshape(K_CT, -1)
    tidx = jnp.transpose(twoopt_idx.astype(jnp.int32) + offs, (2, 0, 1)).reshape(K_CT, -1)
    ctidx = jnp.concatenate([cidx, tidx], axis=1).reshape(-1)

    table_pk = _pack_table(e_emb.reshape(B * E, H))
    xs_c = _sc_gather(table_pk, ctidx, K_CT, M_CTF, 0, 2)
    xs_t = _sc_gather(table_pk, ctidx, K_CT, M_CTF, 1, 2)
    xs_r1 = _sc_gather(table_pk, ridx, K_R, M_RF, 0, 2)
    xs_r2 = _sc_gather(table_pk, ridx, K_R, M_RF, 1, 2)

    r1 = lambda v: v.reshape(1, -1)
    wb = lambda w: w.astype(bf16)
    wacat = jnp.concatenate([wb(Wa1), wb(Wa2), wb(Wa3)], axis=0)
    bacat = jnp.stack([ba1, ba2, ba3], axis=0)
    args_r = (wb(Wr1), r1(br1),
              jnp.concatenate([wb(Wr2), wacat], axis=0),
              jnp.concatenate([r1(br2), bacat], axis=0),
              wb(Wa4).T, r1(ba4))
    args_c = (wb(Wc1), r1(bc1),
              jnp.concatenate([wb(Wc2), wacat], axis=0),
              jnp.concatenate([r1(bc2), bacat], axis=0),
              wb(Wa4).T, r1(ba4))
    lc = _mlp_stack(xs_c, *args_c, row_block=2048, packed=True)
    lt = _mlp_stack(xs_t, *args_c, row_block=2048, packed=True)
    lr1 = _mlp_stack(xs_r1, *args_r, row_block=1024, packed=True)
    lr2 = _mlp_stack(xs_r2, *args_r, row_block=1024, packed=True)

    lr = jnp.concatenate([lr1, lr2], axis=0)
    return jnp.concatenate([lr, lc, lt], axis=1)
